# baseline (speedup 1.0000x reference)
import functools

import jax
import jax.numpy as jnp
from jax import lax
from jax.experimental import pallas as pl
from jax.experimental.pallas import tpu as pltpu
from jax.experimental.pallas import tpu_sc as plsc

_N = 10000
_E = 320000
_D = 128
_H = 256
_G = 64
_NB = 10
_BR = _N // _NB
_K = 80
_NSUB = 16
_NBK = 32
_BW = 320
_CAP = 12000
_NCH = _CAP // _K
_AR = _BW + 8


def _sc_segment_sum(tab, src_flat, dstl_flat, first):
    mesh = plsc.VectorSubcoreMesh(core_axis_name="c", subcore_axis_name="s")
    out_shape = ((_N, 128) if first else (2, _N, 128))

    @functools.partial(
        pl.kernel,
        mesh=mesh,
        out_type=jax.ShapeDtypeStruct(out_shape, jnp.float32),
        scratch_types=[
            pltpu.VMEM((_K,), jnp.int32),
            pltpu.VMEM((_K,), jnp.int32),
            pltpu.VMEM((_K, 128), jnp.float32),
            pltpu.VMEM((_AR, 128), jnp.float32),
            pltpu.VMEM_SHARED((_NSUB * _AR, 128), jnp.float32),
            pltpu.SemaphoreType.DMA,
        ],
    )
    def k(tab_hbm, src_hbm, dst_hbm, out_hbm, src_v, dst_v, rows_v, zbuf,
          acc, sem):
        c = lax.axis_index("c")
        s = lax.axis_index("s")

        @pl.loop(0, _AR)
        def _(i):
            for q in range(8):
                zbuf[i, pl.ds(q * 16, 16)] = jnp.zeros((16,), jnp.float32)

        def reduce_bucket(b):
            pltpu.sync_copy(zbuf, acc.at[pl.ds(s * _AR, _AR)])

            @pl.loop(0, _NCH)
            def _(t):
                off = b * _CAP + t * _K
                pltpu.sync_copy(src_hbm.at[pl.ds(off, _K)], src_v)
                pltpu.sync_copy(dst_hbm.at[pl.ds(off, _K)], dst_v)
                if first:
                    pltpu.async_copy(tab_hbm.at[src_v], rows_v, sem).wait()
                else:
                    pltpu.async_copy(tab_hbm.at[c].at[src_v], rows_v,
                                     sem).wait()
                pltpu.sync_copy(rows_v, acc.at[dst_v], add=True)

        def write_bucket(b):
            @pl.when(b < _NBK - 1)
            def _():
                if first:
                    pltpu.sync_copy(acc.at[pl.ds(s * _AR, _BW)],
                                    out_hbm.at[pl.ds(b * _BW, _BW)])
                else:
                    pltpu.sync_copy(acc.at[pl.ds(s * _AR, _BW)],
                                    out_hbm.at[c, pl.ds(b * _BW, _BW)])

            tail = _N - (_NBK - 1) * _BW

            @pl.when(b == _NBK - 1)
            def _():
                if first:
                    pltpu.sync_copy(
                        acc.at[pl.ds(s * _AR, tail)],
                        out_hbm.at[pl.ds(b * _BW, tail)])
                else:
                    pltpu.sync_copy(
                        acc.at[pl.ds(s * _AR, tail)],
                        out_hbm.at[c, pl.ds(b * _BW, tail)])

        if first:
            b = 2 * s + c
            reduce_bucket(b)
            write_bucket(b)
        else:
            reduce_bucket(2 * s)
            write_bucket(2 * s)
            reduce_bucket(2 * s + 1)
            write_bucket(2 * s + 1)

    return k(tab, src_flat, dstl_flat)


def _partition_edges(src, dst):
    bucket = dst // _BW
    onehot = (bucket[:, None] == jnp.arange(_NBK, dtype=jnp.int32)[None, :])
    pos = jnp.cumsum(onehot.astype(jnp.int32), axis=0)
    mypos = jnp.take_along_axis(pos, bucket[:, None], axis=1)[:, 0] - 1
    flat_idx = bucket * _CAP + mypos
    ar = jnp.arange(_NBK * _CAP, dtype=jnp.int32)
    src_flat = ((ar * 37) % _N).at[flat_idx].set(src)
    pad_rows = ((ar // _CAP) // 2) * _AR + _BW + (ar % 8)
    real_rows = (bucket // 2) * _AR + (dst - bucket * _BW)
    dstl_flat = pad_rows.at[flat_idx].set(real_rows)
    return src_flat, dstl_flat


def _mm_t(a, w):
    return lax.dot_general(a.astype(jnp.bfloat16), w.astype(jnp.bfloat16),
                           (((1,), (1,)), ((), ())),
                           preferred_element_type=jnp.float32)


def _tc_gin_layer(h, agg, W1, b1, W2, b2, gamma, beta, *, first):
    din = _D if first else _H

    def body(h_ref, a_ref, w1_ref, b1_ref, w2_ref, b2_ref, g_ref, be_ref,
             out_ref, z_sc, sum_sc, sq_sc):
        p = pl.program_id(0)
        j = pl.program_id(1)

        @pl.when(p == 0)
        def _():
            if first:
                u = h_ref[...] + a_ref[...]
            else:
                u = (jnp.concatenate([h_ref[0], h_ref[1]], axis=1)
                     + jnp.concatenate([a_ref[0], a_ref[1]], axis=1))
            z = jnp.maximum(_mm_t(u, w1_ref[...]) + b1_ref[...], 0.0)
            z = jnp.maximum(_mm_t(z, w2_ref[...]) + b2_ref[...], 0.0)
            z_sc[pl.ds(j * _BR, _BR), :] = z
            cs = jnp.sum(z, axis=0, keepdims=True)

            @pl.when(j == 0)
            def _():
                sum_sc[...] = cs

            @pl.when(j > 0)
            def _():
                sum_sc[...] = sum_sc[...] + cs

        @pl.when(p == 1)
        def _():
            m = sum_sc[...] * jnp.float32(1e-4)
            z = z_sc[pl.ds(j * _BR, _BR), :]
            dz = z - m
            cq = jnp.sum(dz * dz, axis=0, keepdims=True)

            @pl.when(j == 0)
            def _():
                sq_sc[...] = cq

            @pl.when(j > 0)
            def _():
                sq_sc[...] = sq_sc[...] + cq

        @pl.when(p == 2)
        def _():
            m = sum_sc[...] * jnp.float32(1e-4)
            v = sq_sc[...] * jnp.float32(1e-4)
            z = z_sc[pl.ds(j * _BR, _BR), :]
            hn = g_ref[...] * (z - m) / jnp.sqrt(v + 1e-5) + be_ref[...]
            out_ref[0] = hn[:, :128]
            out_ref[1] = hn[:, 128:]

    if first:
        h_spec = pl.BlockSpec((_BR, _D), lambda p, j: (j, 0))
        a_spec = pl.BlockSpec((_BR, _D), lambda p, j: (j, 0))
    else:
        h_spec = pl.BlockSpec((2, _BR, 128), lambda p, j: (0, j, 0))
        a_spec = pl.BlockSpec((2, _BR, 128), lambda p, j: (0, j, 0))

    return pl.pallas_call(
        body,
        grid=(3, _NB),
        in_specs=[
            h_spec,
            a_spec,
            pl.BlockSpec((_H, din), lambda p, j: (0, 0)),
            pl.BlockSpec((1, _H), lambda p, j: (0, 0)),
            pl.BlockSpec((_H, _H), lambda p, j: (0, 0)),
            pl.BlockSpec((1, _H), lambda p, j: (0, 0)),
            pl.BlockSpec((1, _H), lambda p, j: (0, 0)),
            pl.BlockSpec((1, _H), lambda p, j: (0, 0)),
        ],
        out_specs=pl.BlockSpec((2, _BR, 128), lambda p, j: (0, j, 0)),
        out_shape=jax.ShapeDtypeStruct((2, _N, 128), jnp.float32),
        scratch_shapes=[
            pltpu.VMEM((_N, _H), jnp.float32),
            pltpu.VMEM((1, _H), jnp.float32),
            pltpu.VMEM((1, _H), jnp.float32),
        ],
    )(h, agg, W1, b1, W2, b2, gamma, beta)


def _tc_pool(h, batch3, att_W, att_b, fc1_W, fc1_b, bn_g, bn_b, fc2_W, fc2_b):
    def body(h_ref, b_ref, aw_ref, ab_ref, f1w_ref, f1b_ref, bg_ref, bb_ref,
             f2w_ref, f2b_ref, out_ref, p_sc, den_sc, smax_sc):
        p = pl.program_id(0)
        j = pl.program_id(1)
        hcat = jnp.concatenate([h_ref[0], h_ref[1]], axis=1)
        s_row = lax.dot_general(aw_ref[...].astype(jnp.bfloat16),
                                hcat.astype(jnp.bfloat16),
                                (((1,), (1,)), ((), ())),
                                preferred_element_type=jnp.float32)
        s_row = s_row + ab_ref[0, 0]
        gids = lax.broadcasted_iota(jnp.int32, (_G, _BR), 0)
        mask = b_ref[0] == gids
        neg = jnp.float32(-jnp.inf)

        @pl.when(p == 0)
        def _():
            sm = jnp.max(jnp.where(mask, jnp.broadcast_to(s_row, (_G, _BR)),
                                   neg), axis=1, keepdims=True)

            @pl.when(j == 0)
            def _():
                smax_sc[...] = sm

            @pl.when(j > 0)
            def _():
                smax_sc[...] = jnp.maximum(smax_sc[...], sm)

        @pl.when(p == 1)
        def _():
            smn = jnp.max(jnp.where(mask, jnp.broadcast_to(smax_sc[...],
                                                           (_G, _BR)), neg),
                          axis=0, keepdims=True)
            e_row = jnp.exp(s_row - smn)
            d_blk = jnp.sum(jnp.where(mask, jnp.broadcast_to(e_row, (_G, _BR)),
                                      0.0), axis=1, keepdims=True)

            @pl.when(j == 0)
            def _():
                den_sc[...] = d_blk

            @pl.when(j > 0)
            def _():
                den_sc[...] = den_sc[...] + d_blk

        @pl.when(p == 2)
        def _():
            smn = jnp.max(jnp.where(mask, jnp.broadcast_to(smax_sc[...],
                                                           (_G, _BR)), neg),
                          axis=0, keepdims=True)
            dn = jnp.max(jnp.where(mask, jnp.broadcast_to(den_sc[...],
                                                          (_G, _BR)), neg),
                         axis=0, keepdims=True)
            att_row = jnp.exp(s_row - smn) / dn
            w = jnp.where(mask, jnp.broadcast_to(att_row, (_G, _BR)), 0.0)
            p_blk = lax.dot_general(w, hcat, (((1,), (0,)), ((), ())),
                                    precision=lax.Precision.HIGHEST,
                                    preferred_element_type=jnp.float32)

            @pl.when(j == 0)
            def _():
                p_sc[...] = p_blk

            @pl.when(j > 0)
            def _():
                p_sc[...] = p_sc[...] + p_blk

            @pl.when(j == _NB - 1)
            def _():
                pooled = p_sc[...]
                z1 = jnp.maximum(_mm_t(pooled, f1w_ref[...]) + f1b_ref[...],
                                 0.0)
                m = jnp.sum(z1, axis=0, keepdims=True) / _G
                d1 = z1 - m
                v = jnp.sum(d1 * d1, axis=0, keepdims=True) / _G
                zn = bg_ref[...] * d1 / jnp.sqrt(v + 1e-5) + bb_ref[...]
                out_ref[...] = _mm_t(zn, f2w_ref[...]) + f2b_ref[...]

    return pl.pallas_call(
        body,
        grid=(3, _NB),
        in_specs=[
            pl.BlockSpec((2, _BR, 128), lambda p, j: (0, j, 0)),
            pl.BlockSpec((1, 1, _BR), lambda p, j: (j, 0, 0)),
            pl.BlockSpec((1, _H), lambda p, j: (0, 0)),
            pl.BlockSpec((1, 1), lambda p, j: (0, 0)),
            pl.BlockSpec((_H // 4, _H), lambda p, j: (0, 0)),
            pl.BlockSpec((1, _H // 4), lambda p, j: (0, 0)),
            pl.BlockSpec((1, _H // 4), lambda p, j: (0, 0)),
            pl.BlockSpec((1, _H // 4), lambda p, j: (0, 0)),
            pl.BlockSpec((2, _H // 4), lambda p, j: (0, 0)),
            pl.BlockSpec((1, 2), lambda p, j: (0, 0)),
        ],
        out_specs=pl.BlockSpec((_G, 2), lambda p, j: (0, 0)),
        out_shape=jax.ShapeDtypeStruct((_G, 2), jnp.float32),
        scratch_shapes=[
            pltpu.VMEM((_G, _H), jnp.float32),
            pltpu.VMEM((_G, 1), jnp.float32),
            pltpu.VMEM((_G, 1), jnp.float32),
        ],
    )(h, batch3, att_W, att_b, fc1_W, fc1_b, bn_g, bn_b, fc2_W, fc2_b)


def kernel(x, edge_index, edge_attr, batch, gin_params, att_W, att_b,
           fc1_W, fc1_b, bn_g, bn_b, fc2_W, fc2_b):
    del edge_attr
    src = edge_index[0]
    dst = edge_index[1]
    src_flat, dstl_flat = _partition_edges(src, dst)
    h = None
    for l, (W1, b1, W2, b2, gamma, beta) in enumerate(gin_params):
        first = l == 0
        tab = x if first else h
        agg = _sc_segment_sum(tab, src_flat, dstl_flat, first)
        h = _tc_gin_layer(
            x if first else h, agg,
            W1, b1.reshape(1, -1), W2, b2.reshape(1, -1),
            gamma.reshape(1, -1), beta.reshape(1, -1), first=first)
    batch3 = batch.reshape(_NB, 1, _BR)
    return _tc_pool(h, batch3, att_W, att_b.reshape(1, 1),
                    fc1_W, fc1_b.reshape(1, -1), bn_g.reshape(1, -1),
                    bn_b.reshape(1, -1), fc2_W, fc2_b.reshape(1, -1))


# speedup vs baseline: 1.3477x; 1.3477x over previous
import functools

import jax
import jax.numpy as jnp
from jax import lax
from jax.experimental import pallas as pl
from jax.experimental.pallas import tpu as pltpu
from jax.experimental.pallas import tpu_sc as plsc

_N = 10000
_E = 320000
_D = 128
_H = 256
_G = 64
_NB = 10
_BR = _N // _NB
_K = 80
_NSUB = 16
_NBK = 32
_BW = 320
_CAP = 12000
_NCH = _CAP // _K
_AR = _BW + 8


def _sc_segment_sum(tab, src_flat, dstl_flat, first):
    mesh = plsc.VectorSubcoreMesh(core_axis_name="c", subcore_axis_name="s")
    out_shape = ((_N, 128) if first else (2, _N, 128))

    @functools.partial(
        pl.kernel,
        mesh=mesh,
        out_type=jax.ShapeDtypeStruct(out_shape, jnp.float32),
        scratch_types=[
            pltpu.VMEM((_CAP,), jnp.int32),
            pltpu.VMEM((2, _K), jnp.int32),
            pltpu.VMEM((2, _K, 128), jnp.float32),
            pltpu.VMEM((_AR, 128), jnp.float32),
            pltpu.VMEM_SHARED((_NSUB * _AR, 128), jnp.float32),
            pltpu.SemaphoreType.DMA,
            pltpu.SemaphoreType.DMA,
        ],
    )
    def k(tab_hbm, src_hbm, dst_hbm, out_hbm, srcall, dst2, rows2, zbuf,
          acc, gsem, isem):
        c = lax.axis_index("c")
        s = lax.axis_index("s")

        @pl.loop(0, _AR)
        def _(i):
            for q in range(8):
                zbuf[i, pl.ds(q * 16, 16)] = jnp.zeros((16,), jnp.float32)

        def reduce_bucket(b):
            pltpu.sync_copy(zbuf, acc.at[pl.ds(s * _AR, _AR)])
            pltpu.sync_copy(src_hbm.at[pl.ds(b * _CAP, _CAP)], srcall)
            tab = tab_hbm if first else tab_hbm.at[c]

            def issue(t, buf):
                pltpu.async_copy(dst_hbm.at[pl.ds(b * _CAP + t * _K, _K)],
                                 dst2.at[buf], isem)
                pltpu.async_copy(tab.at[srcall.at[pl.ds(t * _K, _K)]],
                                 rows2.at[buf], gsem)

            def drain(buf):
                pltpu.make_async_copy(dst_hbm.at[pl.ds(0, _K)],
                                      dst2.at[buf], isem).wait()
                pltpu.make_async_copy(tab.at[pl.ds(0, _K)],
                                      rows2.at[buf], gsem).wait()

            def scatter(buf):
                pltpu.sync_copy(rows2.at[buf], acc.at[dst2.at[buf]],
                                add=True)

            issue(0, 0)

            @pl.loop(0, _NCH - 2, step=2)
            def _(t):
                issue(t + 1, 1)
                drain(0)
                scatter(0)
                issue(t + 2, 0)
                drain(1)
                scatter(1)

            issue(_NCH - 1, 1)
            drain(0)
            scatter(0)
            drain(1)
            scatter(1)

        def write_bucket(b):
            @pl.when(b < _NBK - 1)
            def _():
                if first:
                    pltpu.sync_copy(acc.at[pl.ds(s * _AR, _BW)],
                                    out_hbm.at[pl.ds(b * _BW, _BW)])
                else:
                    pltpu.sync_copy(acc.at[pl.ds(s * _AR, _BW)],
                                    out_hbm.at[c, pl.ds(b * _BW, _BW)])

            tail = _N - (_NBK - 1) * _BW

            @pl.when(b == _NBK - 1)
            def _():
                if first:
                    pltpu.sync_copy(
                        acc.at[pl.ds(s * _AR, tail)],
                        out_hbm.at[pl.ds(b * _BW, tail)])
                else:
                    pltpu.sync_copy(
                        acc.at[pl.ds(s * _AR, tail)],
                        out_hbm.at[c, pl.ds(b * _BW, tail)])

        if first:
            b = 2 * s + c
            reduce_bucket(b)
            write_bucket(b)
        else:
            reduce_bucket(2 * s)
            write_bucket(2 * s)
            reduce_bucket(2 * s + 1)
            write_bucket(2 * s + 1)

    return k(tab, src_flat, dstl_flat)


def _partition_edges(src, dst):
    bucket = dst // _BW
    onehot = (bucket[:, None] == jnp.arange(_NBK, dtype=jnp.int32)[None, :])
    pos = jnp.cumsum(onehot.astype(jnp.int32), axis=0)
    mypos = jnp.take_along_axis(pos, bucket[:, None], axis=1)[:, 0] - 1
    flat_idx = bucket * _CAP + mypos
    ar = jnp.arange(_NBK * _CAP, dtype=jnp.int32)
    src_flat = ((ar * 37) % _N).at[flat_idx].set(src)
    pad_rows = ((ar // _CAP) // 2) * _AR + _BW + (ar % 8)
    real_rows = (bucket // 2) * _AR + (dst - bucket * _BW)
    dstl_flat = pad_rows.at[flat_idx].set(real_rows)
    return src_flat, dstl_flat


def _mm_t(a, w):
    return lax.dot_general(a.astype(jnp.bfloat16), w.astype(jnp.bfloat16),
                           (((1,), (1,)), ((), ())),
                           preferred_element_type=jnp.float32)


def _tc_gin_layer(h, agg, W1, b1, W2, b2, gamma, beta, *, first):
    din = _D if first else _H

    def body(h_ref, a_ref, w1_ref, b1_ref, w2_ref, b2_ref, g_ref, be_ref,
             out_ref, z_sc, sum_sc, sq_sc):
        p = pl.program_id(0)
        j = pl.program_id(1)

        @pl.when(p == 0)
        def _():
            if first:
                u = h_ref[...] + a_ref[...]
            else:
                u = (jnp.concatenate([h_ref[0], h_ref[1]], axis=1)
                     + jnp.concatenate([a_ref[0], a_ref[1]], axis=1))
            z = jnp.maximum(_mm_t(u, w1_ref[...]) + b1_ref[...], 0.0)
            z = jnp.maximum(_mm_t(z, w2_ref[...]) + b2_ref[...], 0.0)
            z_sc[pl.ds(j * _BR, _BR), :] = z
            cs = jnp.sum(z, axis=0, keepdims=True)

            @pl.when(j == 0)
            def _():
                sum_sc[...] = cs

            @pl.when(j > 0)
            def _():
                sum_sc[...] = sum_sc[...] + cs

        @pl.when(p == 1)
        def _():
            m = sum_sc[...] * jnp.float32(1e-4)
            z = z_sc[pl.ds(j * _BR, _BR), :]
            dz = z - m
            cq = jnp.sum(dz * dz, axis=0, keepdims=True)

            @pl.when(j == 0)
            def _():
                sq_sc[...] = cq

            @pl.when(j > 0)
            def _():
                sq_sc[...] = sq_sc[...] + cq

        @pl.when(p == 2)
        def _():
            m = sum_sc[...] * jnp.float32(1e-4)
            v = sq_sc[...] * jnp.float32(1e-4)
            z = z_sc[pl.ds(j * _BR, _BR), :]
            hn = g_ref[...] * (z - m) / jnp.sqrt(v + 1e-5) + be_ref[...]
            out_ref[0] = hn[:, :128]
            out_ref[1] = hn[:, 128:]

    if first:
        h_spec = pl.BlockSpec((_BR, _D), lambda p, j: (j, 0))
        a_spec = pl.BlockSpec((_BR, _D), lambda p, j: (j, 0))
    else:
        h_spec = pl.BlockSpec((2, _BR, 128), lambda p, j: (0, j, 0))
        a_spec = pl.BlockSpec((2, _BR, 128), lambda p, j: (0, j, 0))

    return pl.pallas_call(
        body,
        grid=(3, _NB),
        in_specs=[
            h_spec,
            a_spec,
            pl.BlockSpec((_H, din), lambda p, j: (0, 0)),
            pl.BlockSpec((1, _H), lambda p, j: (0, 0)),
            pl.BlockSpec((_H, _H), lambda p, j: (0, 0)),
            pl.BlockSpec((1, _H), lambda p, j: (0, 0)),
            pl.BlockSpec((1, _H), lambda p, j: (0, 0)),
            pl.BlockSpec((1, _H), lambda p, j: (0, 0)),
        ],
        out_specs=pl.BlockSpec((2, _BR, 128), lambda p, j: (0, j, 0)),
        out_shape=jax.ShapeDtypeStruct((2, _N, 128), jnp.float32),
        scratch_shapes=[
            pltpu.VMEM((_N, _H), jnp.float32),
            pltpu.VMEM((1, _H), jnp.float32),
            pltpu.VMEM((1, _H), jnp.float32),
        ],
    )(h, agg, W1, b1, W2, b2, gamma, beta)


def _tc_pool(h, batch3, att_W, att_b, fc1_W, fc1_b, bn_g, bn_b, fc2_W, fc2_b):
    def body(h_ref, b_ref, aw_ref, ab_ref, f1w_ref, f1b_ref, bg_ref, bb_ref,
             f2w_ref, f2b_ref, out_ref, p_sc, den_sc, smax_sc):
        p = pl.program_id(0)
        j = pl.program_id(1)
        hcat = jnp.concatenate([h_ref[0], h_ref[1]], axis=1)
        s_row = lax.dot_general(aw_ref[...].astype(jnp.bfloat16),
                                hcat.astype(jnp.bfloat16),
                                (((1,), (1,)), ((), ())),
                                preferred_element_type=jnp.float32)
        s_row = s_row + ab_ref[0, 0]
        gids = lax.broadcasted_iota(jnp.int32, (_G, _BR), 0)
        mask = b_ref[0] == gids
        neg = jnp.float32(-jnp.inf)

        @pl.when(p == 0)
        def _():
            sm = jnp.max(jnp.where(mask, jnp.broadcast_to(s_row, (_G, _BR)),
                                   neg), axis=1, keepdims=True)

            @pl.when(j == 0)
            def _():
                smax_sc[...] = sm

            @pl.when(j > 0)
            def _():
                smax_sc[...] = jnp.maximum(smax_sc[...], sm)

        @pl.when(p == 1)
        def _():
            smn = jnp.max(jnp.where(mask, jnp.broadcast_to(smax_sc[...],
                                                           (_G, _BR)), neg),
                          axis=0, keepdims=True)
            e_row = jnp.exp(s_row - smn)
            d_blk = jnp.sum(jnp.where(mask, jnp.broadcast_to(e_row, (_G, _BR)),
                                      0.0), axis=1, keepdims=True)

            @pl.when(j == 0)
            def _():
                den_sc[...] = d_blk

            @pl.when(j > 0)
            def _():
                den_sc[...] = den_sc[...] + d_blk

        @pl.when(p == 2)
        def _():
            smn = jnp.max(jnp.where(mask, jnp.broadcast_to(smax_sc[...],
                                                           (_G, _BR)), neg),
                          axis=0, keepdims=True)
            dn = jnp.max(jnp.where(mask, jnp.broadcast_to(den_sc[...],
                                                          (_G, _BR)), neg),
                         axis=0, keepdims=True)
            att_row = jnp.exp(s_row - smn) / dn
            w = jnp.where(mask, jnp.broadcast_to(att_row, (_G, _BR)), 0.0)
            p_blk = lax.dot_general(w, hcat, (((1,), (0,)), ((), ())),
                                    precision=lax.Precision.HIGHEST,
                                    preferred_element_type=jnp.float32)

            @pl.when(j == 0)
            def _():
                p_sc[...] = p_blk

            @pl.when(j > 0)
            def _():
                p_sc[...] = p_sc[...] + p_blk

            @pl.when(j == _NB - 1)
            def _():
                pooled = p_sc[...]
                z1 = jnp.maximum(_mm_t(pooled, f1w_ref[...]) + f1b_ref[...],
                                 0.0)
                m = jnp.sum(z1, axis=0, keepdims=True) / _G
                d1 = z1 - m
                v = jnp.sum(d1 * d1, axis=0, keepdims=True) / _G
                zn = bg_ref[...] * d1 / jnp.sqrt(v + 1e-5) + bb_ref[...]
                out_ref[...] = _mm_t(zn, f2w_ref[...]) + f2b_ref[...]

    return pl.pallas_call(
        body,
        grid=(3, _NB),
        in_specs=[
            pl.BlockSpec((2, _BR, 128), lambda p, j: (0, j, 0)),
            pl.BlockSpec((1, 1, _BR), lambda p, j: (j, 0, 0)),
            pl.BlockSpec((1, _H), lambda p, j: (0, 0)),
            pl.BlockSpec((1, 1), lambda p, j: (0, 0)),
            pl.BlockSpec((_H // 4, _H), lambda p, j: (0, 0)),
            pl.BlockSpec((1, _H // 4), lambda p, j: (0, 0)),
            pl.BlockSpec((1, _H // 4), lambda p, j: (0, 0)),
            pl.BlockSpec((1, _H // 4), lambda p, j: (0, 0)),
            pl.BlockSpec((2, _H // 4), lambda p, j: (0, 0)),
            pl.BlockSpec((1, 2), lambda p, j: (0, 0)),
        ],
        out_specs=pl.BlockSpec((_G, 2), lambda p, j: (0, 0)),
        out_shape=jax.ShapeDtypeStruct((_G, 2), jnp.float32),
        scratch_shapes=[
            pltpu.VMEM((_G, _H), jnp.float32),
            pltpu.VMEM((_G, 1), jnp.float32),
            pltpu.VMEM((_G, 1), jnp.float32),
        ],
    )(h, batch3, att_W, att_b, fc1_W, fc1_b, bn_g, bn_b, fc2_W, fc2_b)


def kernel(x, edge_index, edge_attr, batch, gin_params, att_W, att_b,
           fc1_W, fc1_b, bn_g, bn_b, fc2_W, fc2_b):
    del edge_attr
    src = edge_index[0]
    dst = edge_index[1]
    src_flat, dstl_flat = _partition_edges(src, dst)
    h = None
    for l, (W1, b1, W2, b2, gamma, beta) in enumerate(gin_params):
        first = l == 0
        tab = x if first else h
        agg = _sc_segment_sum(tab, src_flat, dstl_flat, first)
        h = _tc_gin_layer(
            x if first else h, agg,
            W1, b1.reshape(1, -1), W2, b2.reshape(1, -1),
            gamma.reshape(1, -1), beta.reshape(1, -1), first=first)
    batch3 = batch.reshape(_NB, 1, _BR)
    return _tc_pool(h, batch3, att_W, att_b.reshape(1, 1),
                    fc1_W, fc1_b.reshape(1, -1), bn_g.reshape(1, -1),
                    bn_b.reshape(1, -1), fc2_W, fc2_b.reshape(1, -1))


# speedup vs baseline: 1.3791x; 1.0233x over previous
import functools

import jax
import jax.numpy as jnp
from jax import lax
from jax.experimental import pallas as pl
from jax.experimental.pallas import tpu as pltpu
from jax.experimental.pallas import tpu_sc as plsc

_N = 10000
_E = 320000
_D = 128
_H = 256
_G = 64
_NB = 10
_BR = _N // _NB
_K = 128
_NSUB = 16
_NBK = 32
_BW = 320
_CAP = 12032
_NCH = _CAP // _K
_AR = _BW + 8


def _sc_segment_sum(tab, src_flat, dstl_flat, first):
    mesh = plsc.VectorSubcoreMesh(core_axis_name="c", subcore_axis_name="s")
    out_shape = ((_N, 128) if first else (2, _N, 128))

    @functools.partial(
        pl.kernel,
        mesh=mesh,
        out_type=jax.ShapeDtypeStruct(out_shape, jnp.float32),
        scratch_types=[
            pltpu.VMEM((_CAP,), jnp.int32),
            pltpu.VMEM((2, _K), jnp.int32),
            pltpu.VMEM((2, _K, 128), jnp.float32),
            pltpu.VMEM((_AR, 128), jnp.float32),
            pltpu.VMEM_SHARED((_NSUB * _AR, 128), jnp.float32),
            pltpu.SemaphoreType.DMA,
            pltpu.SemaphoreType.DMA,
        ],
    )
    def k(tab_hbm, src_hbm, dst_hbm, out_hbm, srcall, dst2, rows2, zbuf,
          acc, gsem, isem):
        c = lax.axis_index("c")
        s = lax.axis_index("s")

        @pl.loop(0, _AR)
        def _(i):
            for q in range(8):
                zbuf[i, pl.ds(q * 16, 16)] = jnp.zeros((16,), jnp.float32)

        def reduce_bucket(b):
            pltpu.sync_copy(zbuf, acc.at[pl.ds(s * _AR, _AR)])
            pltpu.sync_copy(src_hbm.at[pl.ds(b * _CAP, _CAP)], srcall)
            tab = tab_hbm if first else tab_hbm.at[c]

            def issue(t, buf):
                pltpu.async_copy(dst_hbm.at[pl.ds(b * _CAP + t * _K, _K)],
                                 dst2.at[buf], isem)
                pltpu.async_copy(tab.at[srcall.at[pl.ds(t * _K, _K)]],
                                 rows2.at[buf], gsem)

            def drain(buf):
                pltpu.make_async_copy(dst_hbm.at[pl.ds(0, _K)],
                                      dst2.at[buf], isem).wait()
                pltpu.make_async_copy(tab.at[pl.ds(0, _K)],
                                      rows2.at[buf], gsem).wait()

            def scatter(buf):
                pltpu.sync_copy(rows2.at[buf], acc.at[dst2.at[buf]],
                                add=True)

            issue(0, 0)

            @pl.loop(0, _NCH - 2, step=2)
            def _(t):
                issue(t + 1, 1)
                drain(0)
                scatter(0)
                issue(t + 2, 0)
                drain(1)
                scatter(1)

            issue(_NCH - 1, 1)
            drain(0)
            scatter(0)
            drain(1)
            scatter(1)

        def write_bucket(b):
            @pl.when(b < _NBK - 1)
            def _():
                if first:
                    pltpu.sync_copy(acc.at[pl.ds(s * _AR, _BW)],
                                    out_hbm.at[pl.ds(b * _BW, _BW)])
                else:
                    pltpu.sync_copy(acc.at[pl.ds(s * _AR, _BW)],
                                    out_hbm.at[c, pl.ds(b * _BW, _BW)])

            tail = _N - (_NBK - 1) * _BW

            @pl.when(b == _NBK - 1)
            def _():
                if first:
                    pltpu.sync_copy(
                        acc.at[pl.ds(s * _AR, tail)],
                        out_hbm.at[pl.ds(b * _BW, tail)])
                else:
                    pltpu.sync_copy(
                        acc.at[pl.ds(s * _AR, tail)],
                        out_hbm.at[c, pl.ds(b * _BW, tail)])

        if first:
            b = 2 * s + c
            reduce_bucket(b)
            write_bucket(b)
        else:
            reduce_bucket(2 * s)
            write_bucket(2 * s)
            reduce_bucket(2 * s + 1)
            write_bucket(2 * s + 1)

    return k(tab, src_flat, dstl_flat)


def _partition_edges(src, dst):
    bucket = dst // _BW
    onehot = (bucket[:, None] == jnp.arange(_NBK, dtype=jnp.int32)[None, :])
    pos = jnp.cumsum(onehot.astype(jnp.int32), axis=0)
    mypos = jnp.take_along_axis(pos, bucket[:, None], axis=1)[:, 0] - 1
    flat_idx = bucket * _CAP + mypos
    ar = jnp.arange(_NBK * _CAP, dtype=jnp.int32)
    src_flat = ((ar * 37) % _N).at[flat_idx].set(src)
    pad_rows = ((ar // _CAP) // 2) * _AR + _BW + (ar % 8)
    real_rows = (bucket // 2) * _AR + (dst - bucket * _BW)
    dstl_flat = pad_rows.at[flat_idx].set(real_rows)
    return src_flat, dstl_flat


def _mm_t(a, w):
    return lax.dot_general(a.astype(jnp.bfloat16), w.astype(jnp.bfloat16),
                           (((1,), (1,)), ((), ())),
                           preferred_element_type=jnp.float32)


def _tc_gin_layer(h, agg, W1, b1, W2, b2, gamma, beta, *, first):
    din = _D if first else _H

    def body(h_ref, a_ref, w1_ref, b1_ref, w2_ref, b2_ref, g_ref, be_ref,
             out_ref, z_sc, sum_sc, sq_sc):
        p = pl.program_id(0)
        j = pl.program_id(1)

        @pl.when(p == 0)
        def _():
            if first:
                u = h_ref[...] + a_ref[...]
            else:
                u = (jnp.concatenate([h_ref[0], h_ref[1]], axis=1)
                     + jnp.concatenate([a_ref[0], a_ref[1]], axis=1))
            z = jnp.maximum(_mm_t(u, w1_ref[...]) + b1_ref[...], 0.0)
            z = jnp.maximum(_mm_t(z, w2_ref[...]) + b2_ref[...], 0.0)
            z_sc[pl.ds(j * _BR, _BR), :] = z
            cs = jnp.sum(z, axis=0, keepdims=True)

            @pl.when(j == 0)
            def _():
                sum_sc[...] = cs

            @pl.when(j > 0)
            def _():
                sum_sc[...] = sum_sc[...] + cs

        @pl.when(p == 1)
        def _():
            m = sum_sc[...] * jnp.float32(1e-4)
            z = z_sc[pl.ds(j * _BR, _BR), :]
            dz = z - m
            cq = jnp.sum(dz * dz, axis=0, keepdims=True)

            @pl.when(j == 0)
            def _():
                sq_sc[...] = cq

            @pl.when(j > 0)
            def _():
                sq_sc[...] = sq_sc[...] + cq

        @pl.when(p == 2)
        def _():
            m = sum_sc[...] * jnp.float32(1e-4)
            v = sq_sc[...] * jnp.float32(1e-4)
            z = z_sc[pl.ds(j * _BR, _BR), :]
            hn = g_ref[...] * (z - m) / jnp.sqrt(v + 1e-5) + be_ref[...]
            out_ref[0] = hn[:, :128]
            out_ref[1] = hn[:, 128:]

    if first:
        h_spec = pl.BlockSpec((_BR, _D), lambda p, j: (j, 0))
        a_spec = pl.BlockSpec((_BR, _D), lambda p, j: (j, 0))
    else:
        h_spec = pl.BlockSpec((2, _BR, 128), lambda p, j: (0, j, 0))
        a_spec = pl.BlockSpec((2, _BR, 128), lambda p, j: (0, j, 0))

    return pl.pallas_call(
        body,
        grid=(3, _NB),
        in_specs=[
            h_spec,
            a_spec,
            pl.BlockSpec((_H, din), lambda p, j: (0, 0)),
            pl.BlockSpec((1, _H), lambda p, j: (0, 0)),
            pl.BlockSpec((_H, _H), lambda p, j: (0, 0)),
            pl.BlockSpec((1, _H), lambda p, j: (0, 0)),
            pl.BlockSpec((1, _H), lambda p, j: (0, 0)),
            pl.BlockSpec((1, _H), lambda p, j: (0, 0)),
        ],
        out_specs=pl.BlockSpec((2, _BR, 128), lambda p, j: (0, j, 0)),
        out_shape=jax.ShapeDtypeStruct((2, _N, 128), jnp.float32),
        scratch_shapes=[
            pltpu.VMEM((_N, _H), jnp.float32),
            pltpu.VMEM((1, _H), jnp.float32),
            pltpu.VMEM((1, _H), jnp.float32),
        ],
    )(h, agg, W1, b1, W2, b2, gamma, beta)


def _tc_pool(h, batch3, att_W, att_b, fc1_W, fc1_b, bn_g, bn_b, fc2_W, fc2_b):
    def body(h_ref, b_ref, aw_ref, ab_ref, f1w_ref, f1b_ref, bg_ref, bb_ref,
             f2w_ref, f2b_ref, out_ref, p_sc, den_sc, smax_sc):
        p = pl.program_id(0)
        j = pl.program_id(1)
        hcat = jnp.concatenate([h_ref[0], h_ref[1]], axis=1)
        s_row = lax.dot_general(aw_ref[...].astype(jnp.bfloat16),
                                hcat.astype(jnp.bfloat16),
                                (((1,), (1,)), ((), ())),
                                preferred_element_type=jnp.float32)
        s_row = s_row + ab_ref[0, 0]
        gids = lax.broadcasted_iota(jnp.int32, (_G, _BR), 0)
        mask = b_ref[0] == gids
        neg = jnp.float32(-jnp.inf)

        @pl.when(p == 0)
        def _():
            sm = jnp.max(jnp.where(mask, jnp.broadcast_to(s_row, (_G, _BR)),
                                   neg), axis=1, keepdims=True)

            @pl.when(j == 0)
            def _():
                smax_sc[...] = sm

            @pl.when(j > 0)
            def _():
                smax_sc[...] = jnp.maximum(smax_sc[...], sm)

        @pl.when(p == 1)
        def _():
            smn = jnp.max(jnp.where(mask, jnp.broadcast_to(smax_sc[...],
                                                           (_G, _BR)), neg),
                          axis=0, keepdims=True)
            e_row = jnp.exp(s_row - smn)
            d_blk = jnp.sum(jnp.where(mask, jnp.broadcast_to(e_row, (_G, _BR)),
                                      0.0), axis=1, keepdims=True)

            @pl.when(j == 0)
            def _():
                den_sc[...] = d_blk

            @pl.when(j > 0)
            def _():
                den_sc[...] = den_sc[...] + d_blk

        @pl.when(p == 2)
        def _():
            smn = jnp.max(jnp.where(mask, jnp.broadcast_to(smax_sc[...],
                                                           (_G, _BR)), neg),
                          axis=0, keepdims=True)
            dn = jnp.max(jnp.where(mask, jnp.broadcast_to(den_sc[...],
                                                          (_G, _BR)), neg),
                         axis=0, keepdims=True)
            att_row = jnp.exp(s_row - smn) / dn
            w = jnp.where(mask, jnp.broadcast_to(att_row, (_G, _BR)), 0.0)
            p_blk = lax.dot_general(w, hcat, (((1,), (0,)), ((), ())),
                                    precision=lax.Precision.HIGHEST,
                                    preferred_element_type=jnp.float32)

            @pl.when(j == 0)
            def _():
                p_sc[...] = p_blk

            @pl.when(j > 0)
            def _():
                p_sc[...] = p_sc[...] + p_blk

            @pl.when(j == _NB - 1)
            def _():
                pooled = p_sc[...]
                z1 = jnp.maximum(_mm_t(pooled, f1w_ref[...]) + f1b_ref[...],
                                 0.0)
                m = jnp.sum(z1, axis=0, keepdims=True) / _G
                d1 = z1 - m
                v = jnp.sum(d1 * d1, axis=0, keepdims=True) / _G
                zn = bg_ref[...] * d1 / jnp.sqrt(v + 1e-5) + bb_ref[...]
                out_ref[...] = _mm_t(zn, f2w_ref[...]) + f2b_ref[...]

    return pl.pallas_call(
        body,
        grid=(3, _NB),
        in_specs=[
            pl.BlockSpec((2, _BR, 128), lambda p, j: (0, j, 0)),
            pl.BlockSpec((1, 1, _BR), lambda p, j: (j, 0, 0)),
            pl.BlockSpec((1, _H), lambda p, j: (0, 0)),
            pl.BlockSpec((1, 1), lambda p, j: (0, 0)),
            pl.BlockSpec((_H // 4, _H), lambda p, j: (0, 0)),
            pl.BlockSpec((1, _H // 4), lambda p, j: (0, 0)),
            pl.BlockSpec((1, _H // 4), lambda p, j: (0, 0)),
            pl.BlockSpec((1, _H // 4), lambda p, j: (0, 0)),
            pl.BlockSpec((2, _H // 4), lambda p, j: (0, 0)),
            pl.BlockSpec((1, 2), lambda p, j: (0, 0)),
        ],
        out_specs=pl.BlockSpec((_G, 2), lambda p, j: (0, 0)),
        out_shape=jax.ShapeDtypeStruct((_G, 2), jnp.float32),
        scratch_shapes=[
            pltpu.VMEM((_G, _H), jnp.float32),
            pltpu.VMEM((_G, 1), jnp.float32),
            pltpu.VMEM((_G, 1), jnp.float32),
        ],
    )(h, batch3, att_W, att_b, fc1_W, fc1_b, bn_g, bn_b, fc2_W, fc2_b)


def kernel(x, edge_index, edge_attr, batch, gin_params, att_W, att_b,
           fc1_W, fc1_b, bn_g, bn_b, fc2_W, fc2_b):
    del edge_attr
    src = edge_index[0]
    dst = edge_index[1]
    src_flat, dstl_flat = _partition_edges(src, dst)
    h = None
    for l, (W1, b1, W2, b2, gamma, beta) in enumerate(gin_params):
        first = l == 0
        tab = x if first else h
        agg = _sc_segment_sum(tab, src_flat, dstl_flat, first)
        h = _tc_gin_layer(
            x if first else h, agg,
            W1, b1.reshape(1, -1), W2, b2.reshape(1, -1),
            gamma.reshape(1, -1), beta.reshape(1, -1), first=first)
    batch3 = batch.reshape(_NB, 1, _BR)
    return _tc_pool(h, batch3, att_W, att_b.reshape(1, 1),
                    fc1_W, fc1_b.reshape(1, -1), bn_g.reshape(1, -1),
                    bn_b.reshape(1, -1), fc2_W, fc2_b.reshape(1, -1))
